# baseline (speedup 1.0000x reference)
import jax
import jax.numpy as jnp
from jax.experimental import pallas as pl
from jax.experimental.pallas import tpu as pltpu


def _make_fused_kernel(H, W, C):
    TWO_C = 2 * C
    THREE_C = 3 * C
    SIX_C = 6 * C

    def _body(x_ref, w_ref, b_ref, o_ref, xp_ref):
        xt = jnp.transpose(x_ref[0].astype(jnp.bfloat16))

        xp_ref[0:1, :, :] = jnp.zeros((1, W + 2, C), jnp.bfloat16)
        xp_ref[H + 1:H + 2, :, :] = jnp.zeros((1, W + 2, C), jnp.bfloat16)
        zcol = jnp.zeros((H, 1, C), jnp.bfloat16)
        xp_ref[1:H + 1, 0:1, :] = zcol
        xp_ref[1:H + 1, W + 1:W + 2, :] = zcol
        xp_ref[1:H + 1, 1:W + 1, :] = xt.reshape(H, W, C)

        xp = xp_ref[...]
        taps = []
        for r in range(3):
            for s in range(3):
                taps.append(xp[r:r + H, s:s + W, :].reshape(H * W, C))
        patches = jnp.concatenate(taps, axis=-1)

        acc0 = jnp.dot(patches[:, :SIX_C], w_ref[0],
                       preferred_element_type=jnp.float32) + b_ref[0]
        acc1 = jnp.dot(patches[:, THREE_C:], w_ref[1],
                       preferred_element_type=jnp.float32) + b_ref[1]

        o_ref[0:1, :, 0:1, :, :] = acc0.astype(o_ref.dtype).reshape(1, H, 1, W, TWO_C)
        o_ref[0:1, :, 1:2, :, :] = acc1.astype(o_ref.dtype).reshape(1, H, 1, W, TWO_C)

    return _body


def kernel(x, w_cat, b_cat):
    n, c, h, w = x.shape
    x3 = x.reshape(n, c, h * w)

    wr = w_cat.reshape(2, 3, 3 * c, 2 * c)
    w6 = jnp.stack([wr[0, 0:2].reshape(6 * c, 2 * c),
                    wr[1, 1:3].reshape(6 * c, 2 * c)]).astype(jnp.bfloat16)
    b2 = b_cat.astype(jnp.float32)

    cost = pl.CostEstimate(
        flops=2 * n * h * w * (6 * c) * (4 * c),
        transcendentals=0,
        bytes_accessed=(n * h * w * c) * 4
        + (2 * (6 * c) * (2 * c) + n * h * 2 * w * 2 * c) * 2,
    )
    out6 = pl.pallas_call(
        _make_fused_kernel(h, w, c),
        out_shape=jax.ShapeDtypeStruct((n, h, 2, w, 2 * c), jnp.float32),
        grid=(n,),
        in_specs=[
            pl.BlockSpec((1, c, h * w), lambda ni: (ni, 0, 0)),
            pl.BlockSpec((2, 6 * c, 2 * c), lambda ni: (0, 0, 0)),
            pl.BlockSpec((2, 1, 2 * c), lambda ni: (0, 0, 0)),
        ],
        out_specs=pl.BlockSpec((1, h, 2, w, 2 * c),
                               lambda ni: (ni, 0, 0, 0, 0)),
        scratch_shapes=[pltpu.VMEM((h + 2, w + 2, c), jnp.bfloat16)],
        compiler_params=pltpu.CompilerParams(
            dimension_semantics=("parallel",)),
        cost_estimate=cost,
    )(x3, w6, b2)

    out_nhwc = out6.reshape(n, 2 * h, 2 * w, c)
    return jnp.transpose(out_nhwc, (0, 3, 1, 2))


# speedup vs baseline: 1.0822x; 1.0822x over previous
import jax
import jax.numpy as jnp
from jax.experimental import pallas as pl
from jax.experimental.pallas import tpu as pltpu


def _make_fused_kernel(H, W, C, TS):
    TWO_C = 2 * C
    THREE_C = 3 * C
    SIX_C = 6 * C
    T = H // TS

    def _body(x_ref, w_ref, b_ref, o_ref, xp_ref, xt_ref):
        xt_ref[...] = jnp.transpose(x_ref[0].astype(jnp.bfloat16))

        zrow = jnp.zeros((1, W, C), jnp.bfloat16)
        zcol = jnp.zeros((TS + 2, 1, C), jnp.bfloat16)

        for t in range(T):
            xp_ref[:, 0:1, :] = zcol
            xp_ref[:, W + 1:W + 2, :] = zcol
            xp_ref[1:TS + 1, 1:W + 1, :] = (
                xt_ref[t * TS * W:(t * TS + TS) * W, :].reshape(TS, W, C))
            if t == 0:
                xp_ref[0:1, 1:W + 1, :] = zrow
            else:
                xp_ref[0:1, 1:W + 1, :] = (
                    xt_ref[(t * TS - 1) * W:t * TS * W, :].reshape(1, W, C))
            if t == T - 1:
                xp_ref[TS + 1:TS + 2, 1:W + 1, :] = zrow
            else:
                xp_ref[TS + 1:TS + 2, 1:W + 1, :] = (
                    xt_ref[(t * TS + TS) * W:(t * TS + TS + 1) * W, :]
                    .reshape(1, W, C))

            xp = xp_ref[...]
            taps = []
            for r in range(3):
                for s in range(3):
                    taps.append(xp[r:r + TS, s:s + W, :].reshape(TS * W, C))
            patches = jnp.concatenate(taps, axis=-1)

            acc0 = jnp.dot(patches[:, :SIX_C], w_ref[0],
                           preferred_element_type=jnp.float32) + b_ref[0]
            acc1 = jnp.dot(patches[:, THREE_C:], w_ref[1],
                           preferred_element_type=jnp.float32) + b_ref[1]

            o_ref[0:1, t * TS:(t + 1) * TS, 0:1, :, :] = (
                acc0.astype(o_ref.dtype).reshape(1, TS, 1, W, TWO_C))
            o_ref[0:1, t * TS:(t + 1) * TS, 1:2, :, :] = (
                acc1.astype(o_ref.dtype).reshape(1, TS, 1, W, TWO_C))

    return _body


def kernel(x, w_cat, b_cat):
    n, c, h, w = x.shape
    x3 = x.reshape(n, c, h * w)

    wr = w_cat.reshape(2, 3, 3 * c, 2 * c)
    w6 = jnp.stack([wr[0, 0:2].reshape(6 * c, 2 * c),
                    wr[1, 1:3].reshape(6 * c, 2 * c)]).astype(jnp.bfloat16)
    b2 = b_cat.astype(jnp.float32)

    ts = 16
    while h % ts:
        ts //= 2

    cost = pl.CostEstimate(
        flops=2 * n * h * w * (6 * c) * (4 * c),
        transcendentals=0,
        bytes_accessed=(n * h * w * c) * 4
        + (2 * (6 * c) * (2 * c) + n * h * 2 * w * 2 * c) * 2,
    )
    out6 = pl.pallas_call(
        _make_fused_kernel(h, w, c, ts),
        out_shape=jax.ShapeDtypeStruct((n, h, 2, w, 2 * c), jnp.bfloat16),
        grid=(n,),
        in_specs=[
            pl.BlockSpec((1, c, h * w), lambda ni: (ni, 0, 0)),
            pl.BlockSpec((2, 6 * c, 2 * c), lambda ni: (0, 0, 0)),
            pl.BlockSpec((2, 1, 2 * c), lambda ni: (0, 0, 0)),
        ],
        out_specs=pl.BlockSpec((1, h, 2, w, 2 * c),
                               lambda ni: (ni, 0, 0, 0, 0)),
        scratch_shapes=[pltpu.VMEM((ts + 2, w + 2, c), jnp.bfloat16),
                        pltpu.VMEM((h * w, c), jnp.bfloat16)],
        compiler_params=pltpu.CompilerParams(
            dimension_semantics=("parallel",)),
        cost_estimate=cost,
    )(x3, w6, b2)

    out_nhwc = out6.reshape(n, 2 * h, 2 * w, c)
    return jnp.transpose(out_nhwc, (0, 3, 1, 2)).astype(jnp.float32)
